# baseline (speedup 1.0000x reference)
import functools

import jax
import jax.numpy as jnp
from jax import lax
from jax.experimental import pallas as pl
from jax.experimental.pallas import tpu as pltpu
from jax.experimental.pallas import tpu_sc as plsc

B, L, D, S = 16, 4096, 768, 128
NC, NS = 2, 16
NW = NC * NS
SPW = S // NW
CH = 16
RD = 4
LANES = 16
KD = D // LANES

_mesh = plsc.VectorSubcoreMesh(core_axis_name="c", subcore_axis_name="s")


@functools.partial(
    pl.kernel,
    mesh=_mesh,
    out_type=jax.ShapeDtypeStruct((B * S * D,), jnp.float32),
    scratch_types=[
        pltpu.VMEM((CH, D), jnp.float32),
        pltpu.VMEM((CH, D), jnp.float32),
        pltpu.VMEM((CH, D), jnp.float32),
        pltpu.VMEM((CH, D), jnp.float32),
        pltpu.VMEM((SPW * D,), jnp.float32),
        pltpu.VMEM((SPW * D,), jnp.float32),
        pltpu.VMEM((B * L + LANES,), jnp.int32),
        pltpu.VMEM((2 * LANES,), jnp.int32),
        pltpu.VMEM((2 * LANES,), jnp.int32),
        pltpu.VMEM((3 * LANES,), jnp.int32),
        pltpu.VMEM((CH,), jnp.int32),
        pltpu.VMEM((CH,), jnp.int32),
        pltpu.VMEM((CH,), jnp.int32),
        pltpu.VMEM((CH,), jnp.int32),
        pltpu.SemaphoreType.DMA,
        pltpu.SemaphoreType.DMA,
        pltpu.SemaphoreType.DMA,
        pltpu.SemaphoreType.DMA,
        pltpu.SemaphoreType.DMA,
        pltpu.SemaphoreType.DMA,
        pltpu.SemaphoreType.DMA,
        pltpu.SemaphoreType.DMA,
    ],
)
def _sc_pool(words, bounds, seg, out, dbuf0, dbuf1, dbuf2, dbuf3, acc0, acc1,
             tball, bnds0, bnds1, bbuf, ix0, ix1, ix2, ix3,
             sd0, sd1, sd2, sd3, st0, st1, so0, so1):
    c = lax.axis_index("c")
    s_idx = lax.axis_index("s")
    w = s_idx * NC + c
    iot = lax.iota(jnp.int32, LANES)
    dbufs, accs = (dbuf0, dbuf1, dbuf2, dbuf3), (acc0, acc1)
    bndss = (bnds0, bnds1)
    ixs = (ix0, ix1, ix2, ix3)
    sds, sts, sos = (sd0, sd1, sd2, sd3), (st0, st1), (so0, so1)
    v0 = SPW * w

    def search(b, bn):
        start = bbuf[pl.ds(b, LANES)][0]
        end = bbuf[pl.ds(LANES + b, LANES)][0]
        tb0 = b * L

        def _bs(i, los_his):
            los, his = los_his
            nlos, nhis = [], []
            for j in range(SPW + 1):
                mid = (los[j] + his[j]) >> 1
                ge = tball[pl.ds(tb0 + mid, LANES)][0] >= v0 + j
                nlos.append(jnp.where(ge, los[j], mid + 1))
                nhis.append(jnp.where(ge, mid, his[j]))
            return tuple(nlos), tuple(nhis)

        los, _ = lax.fori_loop(
            0, 12, _bs,
            (tuple(jnp.int32(0) for _ in range(SPW + 1)),
             tuple(jnp.int32(L) for _ in range(SPW + 1))))
        bvals = [jnp.minimum(jnp.maximum(lo, start), end + 1) for lo in los]
        bvec = jnp.full((LANES,), bvals[SPW], jnp.int32)
        for j in range(SPW):
            bvec = jnp.where(iot == j, bvals[j], bvec)
        bn[pl.ds(0, LANES)] = bvec
        t_lo, t_hi = bvals[0], bvals[SPW]
        n = jnp.where(t_hi > t_lo, (t_hi - t_lo + (CH - 1)) >> 4, 0)
        return t_lo, t_hi, n

    def chunk_start(b, t_lo, t_hi, i, cph):
        base = b * L + t_lo + i * CH
        last = b * L + t_hi - 1
        for g in range(CH // LANES):
            ixs[cph][pl.ds(g * LANES, LANES)] = jnp.minimum(
                base + g * LANES + iot, last)
        pltpu.make_async_copy(words.at[ixs[cph]], dbufs[cph],
                              sds[cph]).start()

    def head_start(b, t_lo, t_hi, n):
        for k in range(RD):
            @pl.when(n > k)
            def _hk(k=k):
                chunk_start(b, t_lo, t_hi, k, k)

    pltpu.sync_copy(bounds, bbuf)
    pltpu.sync_copy(seg.at[pl.ds(0, B * L)], tball.at[pl.ds(0, B * L)])
    lo_c, hi_c, n_c = search(0, bnds0)
    head_start(0, lo_c, hi_c, n_c)

    def _batch(bp, carry):
        for ph in range(2):
            b = bp * 2 + ph
            _, n = carry
            acc = accs[ph]
            bn = bndss[ph]

            @pl.when(b >= 2)
            def _wait_out():
                pltpu.make_async_copy(
                    acc, out.at[pl.ds(0, SPW * D)], sos[ph]).wait()

            bvec = bn[pl.ds(0, LANES)]
            t_lo = bvec[0]
            t_hi = bvec[SPW]

            def _chunkquad(i4, ccarry):
                for cph in range(RD):
                    i = i4 * RD + cph

                    @pl.when(i < n)
                    def _do(i=i, cph=cph):
                        dbuf = dbufs[cph]
                        p = t_lo + i * CH
                        pltpu.make_async_copy(
                            words.at[ixs[cph]], dbuf, sds[cph]).wait()
                        proc_lo = p
                        proc_hi = jnp.minimum(t_hi, p + CH)

                        def _bucket(sloc, scarry):
                            t0 = bn[pl.ds(sloc, LANES)][0]
                            t1 = bn[pl.ds(sloc + 1, LANES)][0]
                            lo_i = jnp.maximum(t0, proc_lo) - p
                            hi_i = jnp.minimum(t1, proc_hi) - p

                            @pl.when(hi_i > lo_i)
                            def _run():
                                def _tok(j, racc):
                                    return tuple(
                                        racc[k] + dbuf[lo_i + j,
                                                       pl.ds(k * LANES, LANES)]
                                        for k in range(KD))

                                racc = lax.fori_loop(
                                    0, hi_i - lo_i, _tok,
                                    tuple(jnp.zeros((LANES,), jnp.float32)
                                          for _ in range(KD)))
                                abase = sloc * D

                                @pl.when(t0 >= p)
                                def _store():
                                    for k in range(KD):
                                        acc[pl.ds(abase + k * LANES,
                                                  LANES)] = racc[k]

                                @pl.when(t0 < p)
                                def _add():
                                    for k in range(KD):
                                        acc[pl.ds(abase + k * LANES,
                                                  LANES)] = (
                                            acc[pl.ds(abase + k * LANES,
                                                      LANES)] + racc[k])

                            return scarry

                        lax.fori_loop(0, SPW, _bucket, 0)

                        @pl.when(i + RD < n)
                        def _prn():
                            chunk_start(b, t_lo, t_hi, i + RD, cph)

                return ccarry

            lax.fori_loop(0, (n + RD - 1) >> 2, _chunkquad, 0)

            lo_n, hi_n, n_n = search(jnp.minimum(b + 1, B - 1),
                                     bndss[1 - ph])
            n_n = jnp.where(b + 1 < B, n_n, 0)
            head_start(b + 1, lo_n, hi_n, n_n)

            onev = jnp.ones((LANES,), jnp.float32)

            def _div(sloc, dcarry):
                t0 = bn[pl.ds(sloc, LANES)][0]
                t1 = bn[pl.ds(sloc + 1, LANES)][0]
                cnt = t1 - t0
                cntf = jnp.maximum(cnt.astype(jnp.float32), 1.0)
                inv = jnp.where(cnt > 0, onev / (onev * cntf),
                                jnp.zeros((LANES,), jnp.float32))
                for k in range(KD):
                    acc[pl.ds(sloc * D + k * LANES, LANES)] = (
                        acc[pl.ds(sloc * D + k * LANES, LANES)] * inv)
                return dcarry

            lax.fori_loop(0, SPW, _div, 0)
            obase = pl.multiple_of((b * S + v0) * D, 16)
            pltpu.make_async_copy(
                acc, out.at[pl.ds(obase, SPW * D)], sos[ph]).start()
            carry = (lo_n, n_n)

        return carry

    lax.fori_loop(0, B // 2, _batch, (lo_c, n_c))
    pltpu.make_async_copy(acc0, out.at[pl.ds(0, SPW * D)], so0).wait()
    pltpu.make_async_copy(acc1, out.at[pl.ds(0, SPW * D)], so1).wait()


def kernel(words_emb, bound_passages, sent2subword):
    bounds_flat = jnp.concatenate([
        bound_passages.T.astype(jnp.int32).reshape(2 * LANES),
        jnp.zeros((LANES,), jnp.int32)])
    seg = sent2subword.astype(jnp.int32).reshape(B * L)
    flat = _sc_pool(words_emb.reshape(B * L, D), bounds_flat, seg)
    return flat.reshape(B, S, D)


# speedup vs baseline: 1.0398x; 1.0398x over previous
import functools

import jax
import jax.numpy as jnp
from jax import lax
from jax.experimental import pallas as pl
from jax.experimental.pallas import tpu as pltpu
from jax.experimental.pallas import tpu_sc as plsc

B, L, D, S = 16, 4096, 768, 128
NC, NS = 2, 16
NW = NC * NS
SPW = S // NW
CH = 32
LANES = 16
KD = D // LANES

_mesh = plsc.VectorSubcoreMesh(core_axis_name="c", subcore_axis_name="s")


@functools.partial(
    pl.kernel,
    mesh=_mesh,
    out_type=jax.ShapeDtypeStruct((B * S * D,), jnp.float32),
    scratch_types=[
        pltpu.VMEM((CH, D), jnp.float32),
        pltpu.VMEM((CH, D), jnp.float32),
        pltpu.VMEM((SPW * D,), jnp.float32),
        pltpu.VMEM((SPW * D,), jnp.float32),
        pltpu.VMEM((L + LANES,), jnp.int32),
        pltpu.VMEM((L + LANES,), jnp.int32),
        pltpu.VMEM((2 * LANES,), jnp.int32),
        pltpu.VMEM((2 * LANES,), jnp.int32),
        pltpu.VMEM((3 * LANES,), jnp.int32),
        pltpu.SemaphoreType.DMA,
        pltpu.SemaphoreType.DMA,
        pltpu.SemaphoreType.DMA,
        pltpu.SemaphoreType.DMA,
        pltpu.SemaphoreType.DMA,
        pltpu.SemaphoreType.DMA,
    ],
)
def _sc_pool(words, bounds, seg, out, dbuf0, dbuf1, acc0, acc1, tbuf0, tbuf1,
             bnds0, bnds1, bbuf, sd0, sd1, st0, st1, so0, so1):
    c = lax.axis_index("c")
    s_idx = lax.axis_index("s")
    w = s_idx * NC + c
    iot = lax.iota(jnp.int32, LANES)
    dbufs, accs = (dbuf0, dbuf1), (acc0, acc1)
    tbufs, bndss = (tbuf0, tbuf1), (bnds0, bnds1)
    sds, sts, sos = (sd0, sd1), (st0, st1), (so0, so1)
    v0 = SPW * w

    def seg_start(b, tb, st):
        pltpu.make_async_copy(
            seg.at[pl.ds(pl.multiple_of(b * L, 16), L)],
            tb.at[pl.ds(0, L)], st).start()

    def seg_wait(tb, st):
        pltpu.make_async_copy(seg.at[pl.ds(0, L)], tb.at[pl.ds(0, L)],
                              st).wait()

    def search(b, tb, bn):
        start = bbuf[pl.ds(b, LANES)][0]
        end = bbuf[pl.ds(LANES + b, LANES)][0]

        def _bs(i, los_his):
            los, his = los_his
            nlos, nhis = [], []
            for j in range(SPW + 1):
                mid = (los[j] + his[j]) >> 1
                ge = tb[pl.ds(mid, LANES)][0] >= v0 + j
                nlos.append(jnp.where(ge, los[j], mid + 1))
                nhis.append(jnp.where(ge, mid, his[j]))
            return tuple(nlos), tuple(nhis)

        los, _ = lax.fori_loop(
            0, 12, _bs,
            (tuple(jnp.int32(0) for _ in range(SPW + 1)),
             tuple(jnp.int32(L) for _ in range(SPW + 1))))
        bvals = [jnp.minimum(jnp.maximum(lo, start), end + 1) for lo in los]
        bvec = jnp.full((LANES,), bvals[SPW], jnp.int32)
        for j in range(SPW):
            bvec = jnp.where(iot == j, bvals[j], bvec)
        bn[pl.ds(0, LANES)] = bvec
        t_lo, t_hi = bvals[0], bvals[SPW]
        a0 = lax.bitwise_and(t_lo, jnp.int32(-16))
        n = jnp.where(t_hi > t_lo, (t_hi - a0 + (CH - 1)) >> 5, 0)
        return a0, n

    def chunk_start(b, a0, i, cph):
        p = pl.multiple_of(jnp.minimum(a0 + i * CH, L - CH), 16)
        pltpu.make_async_copy(words.at[b, pl.ds(p, CH)], dbufs[cph],
                              sds[cph]).start()

    pltpu.sync_copy(bounds, bbuf)
    seg_start(0, tbuf0, st0)
    seg_wait(tbuf0, st0)
    seg_start(1, tbuf1, st1)
    a0_c, n_c = search(0, tbuf0, bnds0)

    @pl.when(n_c > 0)
    def _p0():
        chunk_start(0, a0_c, 0, 0)

    @pl.when(n_c > 1)
    def _p1():
        chunk_start(0, a0_c, 1, 1)

    def _batch(bp, carry):
        for ph in range(2):
            b = bp * 2 + ph
            a0, n = carry
            acc = accs[ph]
            bn = bndss[ph]

            @pl.when(b >= 2)
            def _wait_out():
                pltpu.make_async_copy(
                    acc, out.at[pl.ds(0, SPW * D)], sos[ph]).wait()

            bvec = bn[pl.ds(0, LANES)]
            t_lo = bvec[0]
            t_hi = bvec[SPW]

            def _chunkpair(i2, ccarry):
                for cph in range(2):
                    i = i2 * 2 + cph

                    @pl.when(i < n)
                    def _do(i=i, cph=cph):
                        dbuf = dbufs[cph]
                        p_u = a0 + i * CH
                        p = pl.multiple_of(jnp.minimum(p_u, L - CH), 16)
                        pltpu.make_async_copy(
                            words.at[b, pl.ds(p, CH)], dbuf, sds[cph]).wait()
                        proc_lo = jnp.maximum(t_lo, p_u)
                        proc_hi = jnp.minimum(t_hi, p_u + CH)

                        def _bucket(sloc, scarry):
                            t0 = bn[pl.ds(sloc, LANES)][0]
                            t1 = bn[pl.ds(sloc + 1, LANES)][0]
                            lo_i = jnp.maximum(t0, proc_lo) - p
                            hi_i = jnp.minimum(t1, proc_hi) - p

                            @pl.when(hi_i > lo_i)
                            def _run():
                                def _tok(j, racc):
                                    return tuple(
                                        racc[k] + dbuf[lo_i + j,
                                                       pl.ds(k * LANES, LANES)]
                                        for k in range(KD))

                                racc = lax.fori_loop(
                                    0, hi_i - lo_i, _tok,
                                    tuple(jnp.zeros((LANES,), jnp.float32)
                                          for _ in range(KD)))
                                abase = sloc * D

                                @pl.when(t0 >= p_u)
                                def _store():
                                    for k in range(KD):
                                        acc[pl.ds(abase + k * LANES,
                                                  LANES)] = racc[k]

                                @pl.when(t0 < p_u)
                                def _add():
                                    for k in range(KD):
                                        acc[pl.ds(abase + k * LANES,
                                                  LANES)] = (
                                            acc[pl.ds(abase + k * LANES,
                                                      LANES)] + racc[k])

                            return scarry

                        lax.fori_loop(0, SPW, _bucket, 0)

                        @pl.when(i + 2 < n)
                        def _prn():
                            chunk_start(b, a0, i + 2, cph)

                return ccarry

            lax.fori_loop(0, (n + 1) >> 1, _chunkpair, 0)

            @pl.when(b + 1 < B)
            def _wseg():
                seg_wait(tbufs[1 - ph], sts[1 - ph])

            @pl.when(b + 2 < B)
            def _pseg():
                seg_start(b + 2, tbufs[ph], sts[ph])

            a0_n, n_n = search(jnp.minimum(b + 1, B - 1), tbufs[1 - ph],
                               bndss[1 - ph])
            n_n = jnp.where(b + 1 < B, n_n, 0)

            @pl.when(n_n > 0)
            def _c0():
                chunk_start(b + 1, a0_n, 0, 0)

            @pl.when(n_n > 1)
            def _c1():
                chunk_start(b + 1, a0_n, 1, 1)

            onev = jnp.ones((LANES,), jnp.float32)

            def _div(sloc, dcarry):
                t0 = bn[pl.ds(sloc, LANES)][0]
                t1 = bn[pl.ds(sloc + 1, LANES)][0]
                cnt = t1 - t0
                cntf = jnp.maximum(cnt.astype(jnp.float32), 1.0)
                inv = jnp.where(cnt > 0, onev / (onev * cntf),
                                jnp.zeros((LANES,), jnp.float32))
                for k in range(KD):
                    acc[pl.ds(sloc * D + k * LANES, LANES)] = (
                        acc[pl.ds(sloc * D + k * LANES, LANES)] * inv)
                return dcarry

            lax.fori_loop(0, SPW, _div, 0)
            obase = pl.multiple_of((b * S + v0) * D, 16)
            pltpu.make_async_copy(
                acc, out.at[pl.ds(obase, SPW * D)], sos[ph]).start()
            carry = (a0_n, n_n)

        return carry

    lax.fori_loop(0, B // 2, _batch, (a0_c, n_c))
    pltpu.make_async_copy(acc0, out.at[pl.ds(0, SPW * D)], so0).wait()
    pltpu.make_async_copy(acc1, out.at[pl.ds(0, SPW * D)], so1).wait()


def kernel(words_emb, bound_passages, sent2subword):
    bounds_flat = jnp.concatenate([
        bound_passages.T.astype(jnp.int32).reshape(2 * LANES),
        jnp.zeros((LANES,), jnp.int32)])
    seg = sent2subword.astype(jnp.int32).reshape(B * L)
    flat = _sc_pool(words_emb, bounds_flat, seg)
    return flat.reshape(B, S, D)
